# baseline (speedup 1.0000x reference)
import jax
import jax.numpy as jnp
from jax.experimental import pallas as pl
from jax.experimental.pallas import tpu as pltpu

_N = 4096
_D = 512
_BLK = 512
_NB = _N // _BLK
_M = 2 * _N
_MUL = 2.0
_NUM = 5
_LOG2E = 1.4426950408889634
_NT = (((1,), (1,)), ((), ()))


def _mmd_kernel(ci_ref, cj_ref, src_ref, tgt_ref,
                out_ref, acc_ref, sqc_ref, c_ref):
    step = pl.program_id(0)
    n_steps = pl.num_programs(0)

    @pl.when(step == 0)
    def _prologue():
        acc_ref[...] = jnp.zeros_like(acc_ref)
        colsum = jnp.zeros((1, _D), jnp.float32)
        sqsum = jnp.zeros((1, _D), jnp.float32)
        for b in range(_NB):
            sb = src_ref[b * _BLK:(b + 1) * _BLK, :].astype(jnp.float32)
            tb = tgt_ref[b * _BLK:(b + 1) * _BLK, :].astype(jnp.float32)
            s2 = sb * sb
            t2 = tb * tb
            colsum += (jnp.sum(sb, axis=0, keepdims=True)
                       + jnp.sum(tb, axis=0, keepdims=True))
            sqsum += (jnp.sum(s2, axis=0, keepdims=True)
                      + jnp.sum(t2, axis=0, keepdims=True))
            sqc_ref[0, b] = jnp.sum(s2, axis=1, keepdims=True).reshape(1, _BLK)
            sqc_ref[1, b] = jnp.sum(t2, axis=1, keepdims=True).reshape(1, _BLK)
        sum_sq = jnp.sum(sqsum)
        cs2 = jnp.sum(colsum * colsum)
        sum_l2 = 2.0 * _M * sum_sq - 2.0 * cs2
        bw = sum_l2 / (_M * _M - _M) / (_MUL ** (_NUM // 2))
        beta_max = bw * (_MUL ** (_NUM - 1))
        c = _LOG2E / beta_max
        c_ref[0] = c
        sqc_ref[...] = sqc_ref[...] * (-c)

    i = ci_ref[step]
    j = cj_ref[step]
    ri = pl.multiple_of(i * _BLK, _BLK)
    rj = pl.multiple_of(j * _BLK, _BLK)

    si = src_ref[pl.ds(ri, _BLK), :]
    ti = tgt_ref[pl.ds(ri, _BLK), :]
    sj = src_ref[pl.ds(rj, _BLK), :]
    tj = tgt_ref[pl.ds(rj, _BLK), :]

    c = c_ref[0]
    c2 = 2.0 * c

    def sq_rows(a):
        af = a.astype(jnp.float32)
        return jnp.sum(af * af, axis=1, keepdims=True)

    nsi = sq_rows(si) * (-c)
    nti = sq_rows(ti) * (-c)
    nsj = sqc_ref[0, j]
    ntj = sqc_ref[1, j]

    def ksum(a, b, na, nb):
        g = jax.lax.dot_general(a, b, _NT, preferred_element_type=jnp.float32)
        nsum = na + nb
        arg = g * c2 + nsum
        t = jnp.exp2(arg)
        t2 = t * t
        t4 = t2 * t2
        a = t4 * t4 + 1.0
        b = t4 * a + 1.0
        d = t2 * b + 1.0
        return t2 * d + t

    base = ksum(si, sj, nsi, nsj) + ksum(ti, tj, nti, ntj)
    kst = ksum(si, tj, nsi, ntj)

    @pl.when(i != j)
    def _offdiag():
        kts = ksum(ti, sj, nti, nsj)
        acc_ref[...] += (base - kst) - kts

    @pl.when(i == j)
    def _diag():
        acc_ref[...] += 0.5 * base - kst

    @pl.when(step == n_steps - 1)
    def _epilogue():
        rowsum = jnp.sum(acc_ref[...], axis=1, keepdims=True)
        total = jnp.sum(rowsum, axis=0, keepdims=True)
        out_ref[...] = total * (2.0 / (_N * _N))


def kernel(source, target):
    src16 = source.astype(jnp.bfloat16)
    tgt16 = target.astype(jnp.bfloat16)

    cells = [(i, j) for i in range(_NB) for j in range(i, _NB)]
    ci = jnp.array([c[0] for c in cells], dtype=jnp.int32)
    cj = jnp.array([c[1] for c in cells], dtype=jnp.int32)
    n_cells = len(cells)

    vmem_spec = pl.BlockSpec(memory_space=pltpu.VMEM)
    out = pl.pallas_call(
        _mmd_kernel,
        out_shape=jax.ShapeDtypeStruct((1, 1), jnp.float32),
        grid_spec=pltpu.PrefetchScalarGridSpec(
            num_scalar_prefetch=2,
            grid=(n_cells,),
            in_specs=[vmem_spec, vmem_spec],
            out_specs=pl.BlockSpec((1, 1), lambda s, ci, cj: (0, 0)),
            scratch_shapes=[
                pltpu.VMEM((_BLK, _BLK), jnp.float32),
                pltpu.VMEM((2, _NB, 1, _BLK), jnp.float32),
                pltpu.SMEM((1,), jnp.float32),
            ],
        ),
        compiler_params=pltpu.CompilerParams(
            dimension_semantics=("arbitrary",),
            vmem_limit_bytes=48 * 1024 * 1024,
        ),
        name="mmd_loss",
    )(ci, cj, src16, tgt16)
    return out[0, 0]


# speedup vs baseline: 1.0823x; 1.0823x over previous
import jax
import jax.numpy as jnp
from jax.experimental import pallas as pl
from jax.experimental.pallas import tpu as pltpu

_N = 4096
_D = 512
_BLK = 512
_NB = _N // _BLK
_M = 2 * _N
_MUL = 2.0
_NUM = 5
_LOG2E = 1.4426950408889634
_NT = (((1,), (1,)), ((), ()))


def _mmd_kernel(ci_ref, cj_ref, src_ref, tgt_ref,
                out_ref, acc_ref, sqc_ref, c_ref):
    step = pl.program_id(0)
    n_steps = pl.num_programs(0)

    @pl.when(step == 0)
    def _prologue():
        acc_ref[...] = jnp.zeros_like(acc_ref)
        colsum = jnp.zeros((1, _D), jnp.float32)
        sqsum = jnp.zeros((1, _D), jnp.float32)
        for b in range(_NB):
            sb = src_ref[b * _BLK:(b + 1) * _BLK, :].astype(jnp.float32)
            tb = tgt_ref[b * _BLK:(b + 1) * _BLK, :].astype(jnp.float32)
            s2 = sb * sb
            t2 = tb * tb
            colsum += (jnp.sum(sb, axis=0, keepdims=True)
                       + jnp.sum(tb, axis=0, keepdims=True))
            sqsum += (jnp.sum(s2, axis=0, keepdims=True)
                      + jnp.sum(t2, axis=0, keepdims=True))
            sqc_ref[0, b] = jnp.sum(s2, axis=1, keepdims=True).reshape(1, _BLK)
            sqc_ref[1, b] = jnp.sum(t2, axis=1, keepdims=True).reshape(1, _BLK)
        sum_sq = jnp.sum(sqsum)
        cs2 = jnp.sum(colsum * colsum)
        sum_l2 = 2.0 * _M * sum_sq - 2.0 * cs2
        bw = sum_l2 / (_M * _M - _M) / (_MUL ** (_NUM // 2))
        beta_max = bw * (_MUL ** (_NUM - 1))
        c = _LOG2E / beta_max
        c_ref[0] = c
        sqc_ref[...] = sqc_ref[...] * (-c)

    i = ci_ref[step]
    j = cj_ref[step]
    ri = pl.multiple_of(i * _BLK, _BLK)
    rj = pl.multiple_of(j * _BLK, _BLK)

    si = src_ref[pl.ds(ri, _BLK), :]
    ti = tgt_ref[pl.ds(ri, _BLK), :]
    sj = src_ref[pl.ds(rj, _BLK), :]
    tj = tgt_ref[pl.ds(rj, _BLK), :]

    c = c_ref[0]
    c2 = 2.0 * c

    def sq_rows(a):
        af = a.astype(jnp.float32)
        return jnp.sum(af * af, axis=1, keepdims=True)

    nsi = sq_rows(si) * (-c)
    nti = sq_rows(ti) * (-c)
    nsj = sqc_ref[0, j]
    ntj = sqc_ref[1, j]

    def ksum(a, b, na, nb):
        g = jax.lax.dot_general(a, b, _NT, preferred_element_type=jnp.float32)
        nsum = na + nb
        arg = g * c2 + nsum
        t = jnp.exp2(arg)
        t2 = t * t
        t4 = t2 * t2
        a = t4 * t4 + 1.0
        b = t4 * a + 1.0
        d = t2 * b + 1.0
        return t2 * d + t

    combo = ((ksum(si, sj, nsi, nsj) + ksum(ti, tj, nti, ntj))
             - (ksum(si, tj, nsi, ntj) + ksum(ti, sj, nti, nsj)))
    w = jnp.where(i == j, 0.5, 1.0).astype(jnp.float32)
    acc_ref[...] += w * combo

    @pl.when(step == n_steps - 1)
    def _epilogue():
        rowsum = jnp.sum(acc_ref[...], axis=1, keepdims=True)
        total = jnp.sum(rowsum, axis=0, keepdims=True)
        out_ref[...] = total * (2.0 / (_N * _N))


def kernel(source, target):
    src16 = source.astype(jnp.bfloat16)
    tgt16 = target.astype(jnp.bfloat16)

    cells = [(i, j) for i in range(_NB) for j in range(i, _NB)]
    ci = jnp.array([c[0] for c in cells], dtype=jnp.int32)
    cj = jnp.array([c[1] for c in cells], dtype=jnp.int32)
    n_cells = len(cells)

    vmem_spec = pl.BlockSpec(memory_space=pltpu.VMEM)
    out = pl.pallas_call(
        _mmd_kernel,
        out_shape=jax.ShapeDtypeStruct((1, 1), jnp.float32),
        grid_spec=pltpu.PrefetchScalarGridSpec(
            num_scalar_prefetch=2,
            grid=(n_cells,),
            in_specs=[vmem_spec, vmem_spec],
            out_specs=pl.BlockSpec((1, 1), lambda s, ci, cj: (0, 0)),
            scratch_shapes=[
                pltpu.VMEM((_BLK, _BLK), jnp.float32),
                pltpu.VMEM((2, _NB, 1, _BLK), jnp.float32),
                pltpu.SMEM((1,), jnp.float32),
            ],
        ),
        compiler_params=pltpu.CompilerParams(
            dimension_semantics=("arbitrary",),
            vmem_limit_bytes=48 * 1024 * 1024,
        ),
        name="mmd_loss",
    )(ci, cj, src16, tgt16)
    return out[0, 0]


# speedup vs baseline: 1.1867x; 1.0965x over previous
import jax
import jax.numpy as jnp
from jax.experimental import pallas as pl
from jax.experimental.pallas import tpu as pltpu

_N = 4096
_D = 512
_BLK = 512
_NB = _N // _BLK
_M = 2 * _N
_MUL = 2.0
_NUM = 5
_LOG2E = 1.4426950408889634
_NT = (((1,), (1,)), ((), ()))


def _mmd_kernel(ci_ref, cj_ref, src_ref, tgt_ref,
                out_ref, acc_ref, sqc_ref, c_ref):
    step = pl.program_id(0)
    n_steps = pl.num_programs(0)

    @pl.when(step == 0)
    def _prologue():
        acc_ref[...] = jnp.zeros_like(acc_ref)
        colsum = jnp.zeros((1, _D), jnp.float32)
        sqsum = jnp.zeros((1, _D), jnp.float32)
        for b in range(_NB):
            sb = src_ref[b * _BLK:(b + 1) * _BLK, :].astype(jnp.float32)
            tb = tgt_ref[b * _BLK:(b + 1) * _BLK, :].astype(jnp.float32)
            s2 = sb * sb
            t2 = tb * tb
            colsum += (jnp.sum(sb, axis=0, keepdims=True)
                       + jnp.sum(tb, axis=0, keepdims=True))
            sqsum += (jnp.sum(s2, axis=0, keepdims=True)
                      + jnp.sum(t2, axis=0, keepdims=True))
            sqc_ref[0, b] = jnp.sum(s2, axis=1, keepdims=True).reshape(1, _BLK)
            sqc_ref[1, b] = jnp.sum(t2, axis=1, keepdims=True).reshape(1, _BLK)
        sum_sq = jnp.sum(sqsum)
        cs2 = jnp.sum(colsum * colsum)
        sum_l2 = 2.0 * _M * sum_sq - 2.0 * cs2
        bw = sum_l2 / (_M * _M - _M) / (_MUL ** (_NUM // 2))
        beta_max = bw * (_MUL ** (_NUM - 1))
        c = _LOG2E / beta_max
        c_ref[0] = c
        sqc_ref[...] = sqc_ref[...] * (-c)

    i = ci_ref[step]
    j = cj_ref[step]
    ri = pl.multiple_of(i * _BLK, _BLK)
    rj = pl.multiple_of(j * _BLK, _BLK)

    si = src_ref[pl.ds(ri, _BLK), :]
    ti = tgt_ref[pl.ds(ri, _BLK), :]
    sj = src_ref[pl.ds(rj, _BLK), :]
    tj = tgt_ref[pl.ds(rj, _BLK), :]

    c = c_ref[0]
    c2 = 2.0 * c

    def sq_rows(a):
        af = a.astype(jnp.float32)
        return jnp.sum(af * af, axis=1, keepdims=True)

    nsi = sq_rows(si) * (-c)
    nti = sq_rows(ti) * (-c)
    nsj = sqc_ref[0, j]
    ntj = sqc_ref[1, j]

    def ksum(a, b, na, nb):
        g = jax.lax.dot_general(a, b, _NT, preferred_element_type=jnp.float32)
        arg = (g * c2 + na) + nb
        t = jnp.exp2(arg)
        t2 = t * t
        t4 = t2 * t2
        t8 = t4 * t4
        t16 = t8 * t8
        return ((t + t2) + (t4 + t8)) + t16

    combo = ((ksum(si, sj, nsi, nsj) + ksum(ti, tj, nti, ntj))
             - (ksum(si, tj, nsi, ntj) + ksum(ti, sj, nti, nsj)))
    w = jnp.where(i == j, 0.5, 1.0).astype(jnp.float32)
    acc_ref[...] += w * combo

    @pl.when(step == n_steps - 1)
    def _epilogue():
        rowsum = jnp.sum(acc_ref[...], axis=1, keepdims=True)
        total = jnp.sum(rowsum, axis=0, keepdims=True)
        out_ref[...] = total * (2.0 / (_N * _N))


def kernel(source, target):
    src16 = source.astype(jnp.bfloat16)
    tgt16 = target.astype(jnp.bfloat16)

    cells = [(i, j) for i in range(_NB) for j in range(i, _NB)]
    ci = jnp.array([c[0] for c in cells], dtype=jnp.int32)
    cj = jnp.array([c[1] for c in cells], dtype=jnp.int32)
    n_cells = len(cells)

    vmem_spec = pl.BlockSpec(memory_space=pltpu.VMEM)
    out = pl.pallas_call(
        _mmd_kernel,
        out_shape=jax.ShapeDtypeStruct((1, 1), jnp.float32),
        grid_spec=pltpu.PrefetchScalarGridSpec(
            num_scalar_prefetch=2,
            grid=(n_cells,),
            in_specs=[vmem_spec, vmem_spec],
            out_specs=pl.BlockSpec((1, 1), lambda s, ci, cj: (0, 0)),
            scratch_shapes=[
                pltpu.VMEM((_BLK, _BLK), jnp.float32),
                pltpu.VMEM((2, _NB, 1, _BLK), jnp.float32),
                pltpu.SMEM((1,), jnp.float32),
            ],
        ),
        compiler_params=pltpu.CompilerParams(
            dimension_semantics=("arbitrary",),
            vmem_limit_bytes=48 * 1024 * 1024,
        ),
        name="mmd_loss",
    )(ci, cj, src16, tgt16)
    return out[0, 0]


# speedup vs baseline: 1.2070x; 1.0171x over previous
import jax
import jax.numpy as jnp
from jax.experimental import pallas as pl
from jax.experimental.pallas import tpu as pltpu

_N = 4096
_D = 512
_BLK = 512
_UB = 2 * _BLK
_NB = _N // _BLK
_M = 2 * _N
_MUL = 2.0
_NUM = 5
_LOG2E = 1.4426950408889634
_NT = (((1,), (1,)), ((), ()))


def _mmd_kernel(ci_ref, cj_ref, src_ref, tgt_ref,
                out_ref, u16_ref, acc_ref, sqc_ref, c_ref):
    step = pl.program_id(0)
    n_steps = pl.num_programs(0)

    @pl.when(step == 0)
    def _prologue():
        acc_ref[...] = jnp.zeros_like(acc_ref)
        colsum = jnp.zeros((1, _D), jnp.float32)
        sqsum = jnp.zeros((1, _D), jnp.float32)
        for b in range(_NB):
            sb = src_ref[b * _BLK:(b + 1) * _BLK, :]
            tb = tgt_ref[b * _BLK:(b + 1) * _BLK, :]
            u16_ref[(2 * b) * _BLK:(2 * b + 1) * _BLK, :] = sb.astype(jnp.bfloat16)
            u16_ref[(2 * b + 1) * _BLK:(2 * b + 2) * _BLK, :] = tb.astype(jnp.bfloat16)
            s2 = sb * sb
            t2 = tb * tb
            colsum += (jnp.sum(sb, axis=0, keepdims=True)
                       + jnp.sum(tb, axis=0, keepdims=True))
            sqsum += (jnp.sum(s2, axis=0, keepdims=True)
                      + jnp.sum(t2, axis=0, keepdims=True))
            rows = jnp.concatenate(
                [jnp.sum(s2, axis=1, keepdims=True),
                 jnp.sum(t2, axis=1, keepdims=True)], axis=0)
            sqc_ref[b] = rows.reshape(1, _UB)
        sum_sq = jnp.sum(sqsum)
        cs2 = jnp.sum(colsum * colsum)
        sum_l2 = 2.0 * _M * sum_sq - 2.0 * cs2
        bw = sum_l2 / (_M * _M - _M) / (_MUL ** (_NUM // 2))
        beta_max = bw * (_MUL ** (_NUM - 1))
        c = _LOG2E / beta_max
        c_ref[0] = c
        sqc_ref[...] = sqc_ref[...] * (-c)

    i = ci_ref[step]
    j = cj_ref[step]
    ri = pl.multiple_of(i * _UB, _UB)
    rj = pl.multiple_of(j * _UB, _UB)

    ui = u16_ref[pl.ds(ri, _UB), :]
    uj = u16_ref[pl.ds(rj, _UB), :]

    c = c_ref[0]
    c2 = 2.0 * c

    uif = ui.astype(jnp.float32)
    nrow = jnp.sum(uif * uif, axis=1, keepdims=True) * (-c)
    ncol = sqc_ref[j]

    g = jax.lax.dot_general(ui, uj, _NT, preferred_element_type=jnp.float32)
    arg = (g * c2 + nrow) + ncol
    t = jnp.exp2(arg)
    t2 = t * t
    t4 = t2 * t2
    t8 = t4 * t4
    t16 = t8 * t8
    ks = ((t + t2) + (t4 + t8)) + t16

    combo = ((ks[:_BLK, :_BLK] + ks[_BLK:, _BLK:])
             - (ks[:_BLK, _BLK:] + ks[_BLK:, :_BLK]))
    w = jnp.where(i == j, 0.5, 1.0).astype(jnp.float32)
    acc_ref[...] += w * combo

    @pl.when(step == n_steps - 1)
    def _epilogue():
        rowsum = jnp.sum(acc_ref[...], axis=1, keepdims=True)
        total = jnp.sum(rowsum, axis=0, keepdims=True)
        out_ref[...] = total * (2.0 / (_N * _N))


def kernel(source, target):
    cells = [(i, j) for i in range(_NB) for j in range(i, _NB)]
    ci = jnp.array([c[0] for c in cells], dtype=jnp.int32)
    cj = jnp.array([c[1] for c in cells], dtype=jnp.int32)
    n_cells = len(cells)

    vmem_spec = pl.BlockSpec(memory_space=pltpu.VMEM)
    out = pl.pallas_call(
        _mmd_kernel,
        out_shape=jax.ShapeDtypeStruct((1, 1), jnp.float32),
        grid_spec=pltpu.PrefetchScalarGridSpec(
            num_scalar_prefetch=2,
            grid=(n_cells,),
            in_specs=[vmem_spec, vmem_spec],
            out_specs=pl.BlockSpec((1, 1), lambda s, ci, cj: (0, 0)),
            scratch_shapes=[
                pltpu.VMEM((_M, _D), jnp.bfloat16),
                pltpu.VMEM((_BLK, _BLK), jnp.float32),
                pltpu.VMEM((_NB, 1, _UB), jnp.float32),
                pltpu.SMEM((1,), jnp.float32),
            ],
        ),
        compiler_params=pltpu.CompilerParams(
            dimension_semantics=("arbitrary",),
            vmem_limit_bytes=48 * 1024 * 1024,
        ),
        name="mmd_loss",
    )(ci, cj, source, target)
    return out[0, 0]


# speedup vs baseline: 1.2146x; 1.0063x over previous
import jax
import jax.numpy as jnp
from jax.experimental import pallas as pl
from jax.experimental.pallas import tpu as pltpu

_N = 4096
_D = 512
_BLK = 512
_UB = 2 * _BLK
_NB = _N // _BLK
_M = 2 * _N
_MUL = 2.0
_NUM = 5
_LOG2E = 1.4426950408889634
_NT = (((1,), (1,)), ((), ()))


def _mmd_kernel(ci_ref, cj_ref, src_ref, tgt_ref,
                out_ref, u16_ref, acc_ref, sqc_ref, c_ref):
    step = pl.program_id(0)
    n_steps = pl.num_programs(0)

    @pl.when(step == 0)
    def _prologue():
        acc_ref[...] = jnp.zeros_like(acc_ref)
        colsum = jnp.zeros((1, _D), jnp.float32)
        sqsum = jnp.zeros((1, _D), jnp.float32)
        for b in range(_NB):
            sb = src_ref[b * _BLK:(b + 1) * _BLK, :]
            tb = tgt_ref[b * _BLK:(b + 1) * _BLK, :]
            colsum += (jnp.sum(sb, axis=0, keepdims=True)
                       + jnp.sum(tb, axis=0, keepdims=True))
            sqsum += (jnp.sum(sb * sb, axis=0, keepdims=True)
                      + jnp.sum(tb * tb, axis=0, keepdims=True))
        sum_sq = jnp.sum(sqsum)
        cs2 = jnp.sum(colsum * colsum)
        sum_l2 = 2.0 * _M * sum_sq - 2.0 * cs2
        bw = sum_l2 / (_M * _M - _M) / (_MUL ** (_NUM // 2))
        beta_max = bw * (_MUL ** (_NUM - 1))
        c = _LOG2E / beta_max
        c_ref[0] = c
        s = jnp.sqrt(2.0 * c)
        for b in range(_NB):
            sb = src_ref[b * _BLK:(b + 1) * _BLK, :] * s
            tb = tgt_ref[b * _BLK:(b + 1) * _BLK, :] * s
            u16_ref[(2 * b) * _BLK:(2 * b + 1) * _BLK, :] = sb.astype(jnp.bfloat16)
            u16_ref[(2 * b + 1) * _BLK:(2 * b + 2) * _BLK, :] = tb.astype(jnp.bfloat16)
            rows = jnp.concatenate(
                [jnp.sum(sb * sb, axis=1, keepdims=True),
                 jnp.sum(tb * tb, axis=1, keepdims=True)], axis=0) * (-0.5)
            sqc_ref[b] = rows.reshape(1, _UB)

    i = ci_ref[step]
    j = cj_ref[step]
    ri = pl.multiple_of(i * _UB, _UB)
    rj = pl.multiple_of(j * _UB, _UB)

    ui = u16_ref[pl.ds(ri, _UB), :]
    uj = u16_ref[pl.ds(rj, _UB), :]
    uif = ui.astype(jnp.float32)
    nrow = jnp.sum(uif * uif, axis=1, keepdims=True) * (-0.5)
    ncol = sqc_ref[j]

    g = jax.lax.dot_general(ui, uj, _NT, preferred_element_type=jnp.float32)
    arg = (g + nrow) + ncol
    t = jnp.exp2(arg)
    t2 = t * t
    t4 = t2 * t2
    t8 = t4 * t4
    t16 = t8 * t8
    ks = ((t + t2) + (t4 + t8)) + t16

    combo = ((ks[:_BLK, :_BLK] + ks[_BLK:, _BLK:])
             - (ks[:_BLK, _BLK:] + ks[_BLK:, :_BLK]))
    w = jnp.where(i == j, 0.5, 1.0).astype(jnp.float32)
    acc_ref[...] += w * combo

    @pl.when(step == n_steps - 1)
    def _epilogue():
        rowsum = jnp.sum(acc_ref[...], axis=1, keepdims=True)
        total = jnp.sum(rowsum, axis=0, keepdims=True)
        out_ref[...] = total * (2.0 / (_N * _N))


def kernel(source, target):
    cells = [(i, j) for i in range(_NB) for j in range(i, _NB)]
    ci = jnp.array([c[0] for c in cells], dtype=jnp.int32)
    cj = jnp.array([c[1] for c in cells], dtype=jnp.int32)
    n_cells = len(cells)

    vmem_spec = pl.BlockSpec(memory_space=pltpu.VMEM)
    out = pl.pallas_call(
        _mmd_kernel,
        out_shape=jax.ShapeDtypeStruct((1, 1), jnp.float32),
        grid_spec=pltpu.PrefetchScalarGridSpec(
            num_scalar_prefetch=2,
            grid=(n_cells,),
            in_specs=[vmem_spec, vmem_spec],
            out_specs=pl.BlockSpec((1, 1), lambda s, ci, cj: (0, 0)),
            scratch_shapes=[
                pltpu.VMEM((_M, _D), jnp.bfloat16),
                pltpu.VMEM((_BLK, _BLK), jnp.float32),
                pltpu.VMEM((_NB, 1, _UB), jnp.float32),
                pltpu.SMEM((1,), jnp.float32),
            ],
        ),
        compiler_params=pltpu.CompilerParams(
            dimension_semantics=("arbitrary",),
            vmem_limit_bytes=48 * 1024 * 1024,
        ),
        name="mmd_loss",
    )(ci, cj, source, target)
    return out[0, 0]


# speedup vs baseline: 1.2461x; 1.0259x over previous
import jax
import jax.numpy as jnp
from jax.experimental import pallas as pl
from jax.experimental.pallas import tpu as pltpu

_N = 4096
_D = 512
_BLK = 512
_UB = 2 * _BLK
_NB = _N // _BLK
_M = 2 * _N
_MUL = 2.0
_NUM = 5
_LOG2E = 1.4426950408889634
_NT = (((1,), (1,)), ((), ()))
_TN = (((0,), (0,)), ((), ()))


def _mmd_kernel(ci_ref, cj_ref, src_ref, tgt_ref,
                out_ref, uT_ref, acc_ref, sqc_ref, sqr_ref, c_ref):
    step = pl.program_id(0)
    n_steps = pl.num_programs(0)

    @pl.when(step == 0)
    def _prologue():
        acc_ref[...] = jnp.zeros_like(acc_ref)
        colsum = jnp.zeros((1, _D), jnp.float32)
        sqsum = jnp.zeros((1, _D), jnp.float32)
        for b in range(_NB):
            sb = src_ref[b * _BLK:(b + 1) * _BLK, :]
            tb = tgt_ref[b * _BLK:(b + 1) * _BLK, :]
            colsum += (jnp.sum(sb, axis=0, keepdims=True)
                       + jnp.sum(tb, axis=0, keepdims=True))
            sqsum += (jnp.sum(sb * sb, axis=0, keepdims=True)
                      + jnp.sum(tb * tb, axis=0, keepdims=True))
        sum_sq = jnp.sum(sqsum)
        cs2 = jnp.sum(colsum * colsum)
        sum_l2 = 2.0 * _M * sum_sq - 2.0 * cs2
        bw = sum_l2 / (_M * _M - _M) / (_MUL ** (_NUM // 2))
        beta_max = bw * (_MUL ** (_NUM - 1))
        c = _LOG2E / beta_max
        c_ref[0] = c
        s = jnp.sqrt(2.0 * c)
        rr = jax.lax.broadcasted_iota(jnp.int32, (_D, _D), 0)
        cc = jax.lax.broadcasted_iota(jnp.int32, (_D, _D), 1)
        eye = jnp.where(rr == cc, 1.0, 0.0).astype(jnp.bfloat16)
        for b in range(_NB):
            sb = src_ref[b * _BLK:(b + 1) * _BLK, :] * s
            tb = tgt_ref[b * _BLK:(b + 1) * _BLK, :] * s
            rows = jnp.concatenate(
                [jnp.sum(sb * sb, axis=1, keepdims=True),
                 jnp.sum(tb * tb, axis=1, keepdims=True)], axis=0) * (-0.5)
            sqr_ref[b * _UB:(b + 1) * _UB, :] = rows
            ub16 = jnp.concatenate([sb, tb], axis=0).astype(jnp.bfloat16)
            tblk = jax.lax.dot_general(eye, ub16, _NT,
                                       preferred_element_type=jnp.float32)
            uT_ref[:, b * _UB:(b + 1) * _UB] = tblk.astype(jnp.bfloat16)
            sqc_ref[b] = jnp.sum(tblk * tblk, axis=0, keepdims=True) * (-0.5)

    i = ci_ref[step]
    j = cj_ref[step]
    ri = pl.multiple_of(i * _UB, _UB)
    rj = pl.multiple_of(j * _UB, _UB)

    uiT = uT_ref[:, pl.ds(ri, _UB)]
    ujT = uT_ref[:, pl.ds(rj, _UB)]
    nrow = sqr_ref[pl.ds(ri, _UB), :]
    ncol = sqc_ref[j]

    g = jax.lax.dot_general(uiT, ujT, _TN, preferred_element_type=jnp.float32)
    arg = (g + nrow) + ncol
    t = jnp.exp2(arg)
    t2 = t * t
    t4 = t2 * t2
    t8 = t4 * t4
    t16 = t8 * t8
    ks = ((t + t2) + (t4 + t8)) + t16

    combo = ((ks[:_BLK, :_BLK] + ks[_BLK:, _BLK:])
             - (ks[:_BLK, _BLK:] + ks[_BLK:, :_BLK]))
    w = jnp.where(i == j, 0.5, 1.0).astype(jnp.float32)
    acc_ref[...] += w * combo

    @pl.when(step == n_steps - 1)
    def _epilogue():
        rowsum = jnp.sum(acc_ref[...], axis=1, keepdims=True)
        total = jnp.sum(rowsum, axis=0, keepdims=True)
        out_ref[...] = total * (2.0 / (_N * _N))


def kernel(source, target):
    cells = [(i, j) for i in range(_NB) for j in range(i, _NB)]
    ci = jnp.array([c[0] for c in cells], dtype=jnp.int32)
    cj = jnp.array([c[1] for c in cells], dtype=jnp.int32)
    n_cells = len(cells)

    vmem_spec = pl.BlockSpec(memory_space=pltpu.VMEM)
    out = pl.pallas_call(
        _mmd_kernel,
        out_shape=jax.ShapeDtypeStruct((1, 1), jnp.float32),
        grid_spec=pltpu.PrefetchScalarGridSpec(
            num_scalar_prefetch=2,
            grid=(n_cells,),
            in_specs=[vmem_spec, vmem_spec],
            out_specs=pl.BlockSpec((1, 1), lambda s, ci, cj: (0, 0)),
            scratch_shapes=[
                pltpu.VMEM((_D, _M), jnp.bfloat16),
                pltpu.VMEM((_BLK, _BLK), jnp.float32),
                pltpu.VMEM((_NB, 1, _UB), jnp.float32),
                pltpu.VMEM((_M, 1), jnp.float32),
                pltpu.SMEM((1,), jnp.float32),
            ],
        ),
        compiler_params=pltpu.CompilerParams(
            dimension_semantics=("arbitrary",),
            vmem_limit_bytes=48 * 1024 * 1024,
        ),
        name="mmd_loss",
    )(ci, cj, source, target)
    return out[0, 0]


# speedup vs baseline: 1.2623x; 1.0130x over previous
import jax
import jax.numpy as jnp
from jax.experimental import pallas as pl
from jax.experimental.pallas import tpu as pltpu

_N = 4096
_D = 512
_BLK = 512
_UB = 2 * _BLK
_NB = _N // _BLK
_M = 2 * _N
_MUL = 2.0
_NUM = 5
_LOG2E = 1.4426950408889634
_NT = (((1,), (1,)), ((), ()))
_TN = (((0,), (0,)), ((), ()))


def _mmd_kernel(ci_ref, cj_ref, src_ref, tgt_ref,
                out_ref, uT_ref, acc_ref, sqc_ref, sqr_ref, c_ref):
    step = pl.program_id(0)
    n_steps = pl.num_programs(0)

    @pl.when(step == 0)
    def _prologue():
        acc_ref[...] = jnp.zeros_like(acc_ref)
        colsum = jnp.zeros((1, _D), jnp.float32)
        sqsum = jnp.zeros((1, _D), jnp.float32)
        for b in range(_NB):
            sb = src_ref[b * _BLK:(b + 1) * _BLK, :]
            tb = tgt_ref[b * _BLK:(b + 1) * _BLK, :]
            colsum += (jnp.sum(sb, axis=0, keepdims=True)
                       + jnp.sum(tb, axis=0, keepdims=True))
            sqsum += (jnp.sum(sb * sb, axis=0, keepdims=True)
                      + jnp.sum(tb * tb, axis=0, keepdims=True))
        sum_sq = jnp.sum(sqsum)
        cs2 = jnp.sum(colsum * colsum)
        sum_l2 = 2.0 * _M * sum_sq - 2.0 * cs2
        bw = sum_l2 / (_M * _M - _M) / (_MUL ** (_NUM // 2))
        beta_max = bw * (_MUL ** (_NUM - 1))
        c = _LOG2E / beta_max
        c_ref[0] = c
        s = jnp.sqrt(2.0 * c)
        rr = jax.lax.broadcasted_iota(jnp.int32, (_D, _D), 0)
        cc = jax.lax.broadcasted_iota(jnp.int32, (_D, _D), 1)
        eye = jnp.where(rr == cc, 1.0, 0.0).astype(jnp.bfloat16)
        for b in range(_NB):
            sb = src_ref[b * _BLK:(b + 1) * _BLK, :] * s
            tb = tgt_ref[b * _BLK:(b + 1) * _BLK, :] * s
            rows = jnp.concatenate(
                [jnp.sum(sb * sb, axis=1, keepdims=True),
                 jnp.sum(tb * tb, axis=1, keepdims=True)], axis=0) * (-0.5)
            sqr_ref[b * _UB:(b + 1) * _UB, :] = rows
            ub16 = jnp.concatenate([sb, tb], axis=0).astype(jnp.bfloat16)
            tblk = jax.lax.dot_general(eye, ub16, _NT,
                                       preferred_element_type=jnp.float32)
            uT_ref[:, b * _UB:(b + 1) * _UB] = tblk.astype(jnp.bfloat16)
            sqc_ref[b] = jnp.sum(tblk * tblk, axis=0, keepdims=True) * (-0.5)

    def cell(idx):
        i = ci_ref[idx]
        j = cj_ref[idx]
        ri = pl.multiple_of(i * _UB, _UB)
        rj = pl.multiple_of(j * _UB, _UB)

        uiT = uT_ref[:, pl.ds(ri, _UB)]
        ujT = uT_ref[:, pl.ds(rj, _UB)]
        nrow = sqr_ref[pl.ds(ri, _UB), :]
        ncol = sqc_ref[j]

        g = jax.lax.dot_general(uiT, ujT, _TN,
                                preferred_element_type=jnp.float32)
        arg = (g + nrow) + ncol
        t = jnp.exp2(arg)
        t2 = t * t
        t4 = t2 * t2
        t8 = t4 * t4
        t16 = t8 * t8
        ks = ((t + t2) + (t4 + t8)) + t16

        combo = ((ks[:_BLK, :_BLK] + ks[_BLK:, _BLK:])
                 - (ks[:_BLK, _BLK:] + ks[_BLK:, :_BLK]))
        w = jnp.where(i == j, 0.5, 1.0).astype(jnp.float32)
        return w * combo

    acc_ref[...] += cell(2 * step) + cell(2 * step + 1)

    @pl.when(step == n_steps - 1)
    def _epilogue():
        rowsum = jnp.sum(acc_ref[...], axis=1, keepdims=True)
        total = jnp.sum(rowsum, axis=0, keepdims=True)
        out_ref[...] = total * (2.0 / (_N * _N))


def kernel(source, target):
    cells = [(i, j) for i in range(_NB) for j in range(i, _NB)]
    ci = jnp.array([c[0] for c in cells], dtype=jnp.int32)
    cj = jnp.array([c[1] for c in cells], dtype=jnp.int32)
    n_cells = len(cells)

    vmem_spec = pl.BlockSpec(memory_space=pltpu.VMEM)
    out = pl.pallas_call(
        _mmd_kernel,
        out_shape=jax.ShapeDtypeStruct((1, 1), jnp.float32),
        grid_spec=pltpu.PrefetchScalarGridSpec(
            num_scalar_prefetch=2,
            grid=(n_cells // 2,),
            in_specs=[vmem_spec, vmem_spec],
            out_specs=pl.BlockSpec((1, 1), lambda s, ci, cj: (0, 0)),
            scratch_shapes=[
                pltpu.VMEM((_D, _M), jnp.bfloat16),
                pltpu.VMEM((_BLK, _BLK), jnp.float32),
                pltpu.VMEM((_NB, 1, _UB), jnp.float32),
                pltpu.VMEM((_M, 1), jnp.float32),
                pltpu.SMEM((1,), jnp.float32),
            ],
        ),
        compiler_params=pltpu.CompilerParams(
            dimension_semantics=("arbitrary",),
            vmem_limit_bytes=48 * 1024 * 1024,
        ),
        name="mmd_loss",
    )(ci, cj, source, target)
    return out[0, 0]


# speedup vs baseline: 1.2847x; 1.0178x over previous
import jax
import jax.numpy as jnp
from jax.experimental import pallas as pl
from jax.experimental.pallas import tpu as pltpu

_N = 4096
_D = 512
_BLK = 512
_UB = 2 * _BLK
_NB = _N // _BLK
_M = 2 * _N
_MUL = 2.0
_NUM = 5
_LOG2E = 1.4426950408889634
_NT = (((1,), (1,)), ((), ()))
_TN = (((0,), (0,)), ((), ()))


def _mmd_kernel(ci_ref, cj_ref, src_ref, tgt_ref,
                out_ref, uT_ref, acc_ref, sqc_ref, sqr_ref, c_ref):
    step = pl.program_id(0)
    n_steps = pl.num_programs(0)

    @pl.when(step == 0)
    def _prologue():
        acc_ref[...] = jnp.zeros_like(acc_ref)
        colsum = jnp.zeros((1, _D), jnp.float32)
        sqsum = jnp.zeros((1, _D), jnp.float32)
        for b in range(_NB):
            sb = src_ref[b * _BLK:(b + 1) * _BLK, :]
            tb = tgt_ref[b * _BLK:(b + 1) * _BLK, :]
            s2 = sb * sb
            t2 = tb * tb
            colsum += (jnp.sum(sb, axis=0, keepdims=True)
                       + jnp.sum(tb, axis=0, keepdims=True))
            sqsum += (jnp.sum(s2, axis=0, keepdims=True)
                      + jnp.sum(t2, axis=0, keepdims=True))
            sqr_ref[b * _UB:(b + 1) * _UB, :] = jnp.concatenate(
                [jnp.sum(s2, axis=1, keepdims=True),
                 jnp.sum(t2, axis=1, keepdims=True)], axis=0)
        sum_sq = jnp.sum(sqsum)
        cs2 = jnp.sum(colsum * colsum)
        sum_l2 = 2.0 * _M * sum_sq - 2.0 * cs2
        bw = sum_l2 / (_M * _M - _M) / (_MUL ** (_NUM // 2))
        beta_max = bw * (_MUL ** (_NUM - 1))
        c = _LOG2E / beta_max
        c_ref[0] = c
        s = jnp.sqrt(2.0 * c)
        sqr_ref[...] = sqr_ref[...] * (-c)
        rr = jax.lax.broadcasted_iota(jnp.int32, (_D, _D), 0)
        cc = jax.lax.broadcasted_iota(jnp.int32, (_D, _D), 1)
        eye = jnp.where(rr == cc, 1.0, 0.0).astype(jnp.bfloat16)
        for b in range(_NB):
            sb = src_ref[b * _BLK:(b + 1) * _BLK, :] * s
            tb = tgt_ref[b * _BLK:(b + 1) * _BLK, :] * s
            ub16 = jnp.concatenate([sb, tb], axis=0).astype(jnp.bfloat16)
            tblk = jax.lax.dot_general(eye, ub16, _NT,
                                       preferred_element_type=jnp.float32)
            uT_ref[:, b * _UB:(b + 1) * _UB] = tblk.astype(jnp.bfloat16)
            sqc_ref[b] = jnp.sum(tblk * tblk, axis=0, keepdims=True) * (-0.5)

    def cell(idx):
        i = ci_ref[idx]
        j = cj_ref[idx]
        ri = pl.multiple_of(i * _UB, _UB)
        rj = pl.multiple_of(j * _UB, _UB)

        uiT = uT_ref[:, pl.ds(ri, _UB)]
        ujT = uT_ref[:, pl.ds(rj, _UB)]
        nrow = sqr_ref[pl.ds(ri, _UB), :]
        ncol = sqc_ref[j]

        g = jax.lax.dot_general(uiT, ujT, _TN,
                                preferred_element_type=jnp.float32)
        arg = (g + nrow) + ncol
        t = jnp.exp2(arg)
        t2 = t * t
        t4 = t2 * t2
        t8 = t4 * t4
        t16 = t8 * t8
        ks = ((t + t2) + (t4 + t8)) + t16

        combo = ((ks[:_BLK, :_BLK] + ks[_BLK:, _BLK:])
                 - (ks[:_BLK, _BLK:] + ks[_BLK:, :_BLK]))
        w = jnp.where(i == j, 0.5, 1.0).astype(jnp.float32)
        return w * combo

    acc_ref[...] += cell(2 * step) + cell(2 * step + 1)

    @pl.when(step == n_steps - 1)
    def _epilogue():
        rowsum = jnp.sum(acc_ref[...], axis=1, keepdims=True)
        total = jnp.sum(rowsum, axis=0, keepdims=True)
        out_ref[...] = total * (2.0 / (_N * _N))


def kernel(source, target):
    cells = [(i, j) for i in range(_NB) for j in range(i, _NB)]
    ci = jnp.array([c[0] for c in cells], dtype=jnp.int32)
    cj = jnp.array([c[1] for c in cells], dtype=jnp.int32)
    n_cells = len(cells)

    vmem_spec = pl.BlockSpec(memory_space=pltpu.VMEM)
    out = pl.pallas_call(
        _mmd_kernel,
        out_shape=jax.ShapeDtypeStruct((1, 1), jnp.float32),
        grid_spec=pltpu.PrefetchScalarGridSpec(
            num_scalar_prefetch=2,
            grid=(n_cells // 2,),
            in_specs=[vmem_spec, vmem_spec],
            out_specs=pl.BlockSpec((1, 1), lambda s, ci, cj: (0, 0)),
            scratch_shapes=[
                pltpu.VMEM((_D, _M), jnp.bfloat16),
                pltpu.VMEM((_BLK, _BLK), jnp.float32),
                pltpu.VMEM((_NB, 1, _UB), jnp.float32),
                pltpu.VMEM((_M, 1), jnp.float32),
                pltpu.SMEM((1,), jnp.float32),
            ],
        ),
        compiler_params=pltpu.CompilerParams(
            dimension_semantics=("arbitrary",),
            vmem_limit_bytes=56 * 1024 * 1024,
        ),
        name="mmd_loss",
    )(ci, cj, source, target)
    return out[0, 0]
